# baseline (speedup 1.0000x reference)
import math
import numpy as np
import jax
import jax.numpy as jnp
from jax.experimental import pallas as pl
from jax.experimental.pallas import tpu as pltpu

IN_CH = 5; P = 8; SFH = 64; EMB = 128; HID = 512; HEADS = 4; FFN = 2048
LAYERS = 2; NSPEC = 6; TOPK = 2; NSHARED = 2; NCLS = 2; ALPHA = 1e-4
B = 2; S = 256; T = B * S; DH = HID // HEADS


def _pos_encoding_np():
    pos = np.arange(S, dtype=np.float32)[:, None]
    div = np.exp(np.arange(0, EMB, 2, dtype=np.float32) * (-math.log(10000.0) / EMB))
    pe = np.zeros((S, EMB), np.float32)
    pe[:, 0::2] = np.sin(pos * div)
    pe[:, 1::2] = np.cos(pos * div)
    return np.tile(pe, (B, 1))


def _bdot(a, b):
    return jax.lax.dot_general(
        a.astype(jnp.bfloat16), b.astype(jnp.bfloat16),
        (((1,), (0,)), ((), ())), preferred_element_type=jnp.float32)


def _bdot_t(a, b):
    return jax.lax.dot_general(
        a.astype(jnp.bfloat16), b.astype(jnp.bfloat16),
        (((1,), (1,)), ((), ())), preferred_element_type=jnp.float32)


def _ln(x, g, b, eps=1e-5):
    m = x.mean(-1, keepdims=True)
    v = ((x - m) ** 2).mean(-1, keepdims=True)
    return (x - m) / jnp.sqrt(v + eps) * g + b


def _embed_kernel(t_ref, w1, b1, w2, b2, pos, pw, pb, out_ref):
    t = jnp.maximum(t_ref[...] @ w1[...] + b1[...], 0.0)
    t = jnp.maximum(t @ w2[...] + b2[...], 0.0)
    t = t + pos[...]
    out_ref[...] = t @ pw[...] + pb[...]


def _attn_kernel(x_ref, qkvw, qkvb, outw, outb, n1g, n1b, spr, shr,
                 y_ref, g_ref, oh_ref, rp_ref):
    bidx = pl.program_id(0)
    x = x_ref[0]
    qkv = _bdot(x, qkvw[...]) + qkvb[...]
    outs = []
    scale = 1.0 / math.sqrt(DH)
    for hd in range(HEADS):
        q = qkv[:, hd * DH:(hd + 1) * DH]
        k = qkv[:, HID + hd * DH: HID + (hd + 1) * DH]
        v = qkv[:, 2 * HID + hd * DH: 2 * HID + (hd + 1) * DH]
        s = _bdot_t(q, k) * scale
        a = jax.nn.softmax(s, axis=-1)
        outs.append(_bdot(a, v))
    o = jnp.concatenate(outs, axis=1)
    att = _bdot(o, outw[...]) + outb[...]
    y = _ln(x + att, n1g[...], n1b[...])
    y_ref[0] = y

    pr = jax.nn.softmax(y @ spr[...], axis=-1)
    p1 = jnp.full((S, 1), -1.0, jnp.float32)
    i1 = jnp.zeros((S, 1), jnp.int32)
    for e in range(NSPEC):
        pe = pr[:, e:e + 1]
        upd = pe > p1
        i1 = jnp.where(upd, e, i1)
        p1 = jnp.where(upd, pe, p1)
    p2 = jnp.full((S, 1), -1.0, jnp.float32)
    i2 = jnp.zeros((S, 1), jnp.int32)
    for e in range(NSPEC):
        pe = pr[:, e:e + 1]
        upd = (pe > p2) & (i1 != e)
        i2 = jnp.where(upd, e, i2)
        p2 = jnp.where(upd, pe, p2)
    den = p1 + p2 + 1e-9
    w1n = p1 / den
    w2n = p2 / den
    spec_g = []
    for e in range(NSPEC):
        ge = jnp.where(i1 == e, w1n, 0.0) + jnp.where(i2 == e, w2n, 0.0)
        spec_g.append(ge)
    sh_p = jax.nn.softmax(y @ shr[...], axis=-1)
    g_ref[0] = jnp.concatenate(spec_g + [sh_p[:, 0:1], sh_p[:, 1:2]], axis=1)

    oh = []
    for e in range(NSPEC):
        oh.append(jnp.sum(((i1 == e) | (i2 == e)).astype(jnp.float32),
                          axis=0, keepdims=True))
    oh_row = jnp.concatenate(oh, axis=1)
    rp_row = jnp.sum(pr, axis=0, keepdims=True)

    @pl.when(bidx == 0)
    def _():
        oh_ref[...] = oh_row
        rp_ref[...] = rp_row

    @pl.when(bidx > 0)
    def _():
        oh_ref[...] += oh_row
        rp_ref[...] += rp_row


def _moe_dense_kernel(y_ref, w1_ref, b1_ref, w2_ref, b2_ref, g_ref, out_ref):
    e = pl.program_id(0)
    z = _bdot(y_ref[...], w1_ref[0]) + b1_ref[0]
    h1 = 0.5 * z * (1.0 + jax.lax.erf(z * (1.0 / math.sqrt(2.0))))
    o = (_bdot(h1, w2_ref[0]) + b2_ref[0]) * g_ref[0]

    @pl.when(e == 0)
    def _():
        out_ref[...] = o

    @pl.when(e > 0)
    def _():
        out_ref[...] += o


def _combine_kernel(y_ref, spec_ref, shar_ref, mng, mnb, n2g, n2b, out_ref):
    y = y_ref[...]
    m = _ln(y + spec_ref[...] + shar_ref[...], mng[...], mnb[...])
    out_ref[...] = _ln(y + m, n2g[...], n2b[...])


def _head_kernel(h_ref, rw, rb, cw, cb, oh_ref, rp_ref,
                 recon_ref, logits_ref, aux_ref):
    h = h_ref[...]
    recon_ref[...] = h @ rw[...] + rb[...]
    pooled = jnp.concatenate(
        [jnp.mean(h[b * S:(b + 1) * S], axis=0, keepdims=True)
         for b in range(B)], axis=0)
    logits_ref[...] = pooled @ cw[...] + cb[...]
    ohm = oh_ref[...] / float(T)
    rpm = rp_ref[...] / float(T)
    aux_ref[...] = jnp.sum(ohm * rpm).reshape(1, 1)


def _full_spec(shape):
    return pl.BlockSpec(shape, lambda *_: (0,) * len(shape))


def _run_embed(t, p):
    pos = jnp.asarray(_pos_encoding_np())
    return pl.pallas_call(
        _embed_kernel,
        out_shape=jax.ShapeDtypeStruct((T, HID), jnp.float32),
    )(t, p['sf_w1'], p['sf_b1'].reshape(1, -1), p['sf_w2'],
      p['sf_b2'].reshape(1, -1), pos, p['proj_w'], p['proj_b'].reshape(1, -1))


def _run_attn(h, L):
    y, g, oh, rp = pl.pallas_call(
        _attn_kernel,
        grid=(B,),
        in_specs=[
            pl.BlockSpec((1, S, HID), lambda b: (b, 0, 0)),
            _full_spec((HID, 3 * HID)), _full_spec((1, 3 * HID)),
            _full_spec((HID, HID)), _full_spec((1, HID)),
            _full_spec((1, HID)), _full_spec((1, HID)),
            _full_spec((HID, NSPEC)), _full_spec((HID, NSHARED)),
        ],
        out_specs=[
            pl.BlockSpec((1, S, HID), lambda b: (b, 0, 0)),
            pl.BlockSpec((1, S, NSPEC + NSHARED), lambda b: (b, 0, 0)),
            _full_spec((1, NSPEC)), _full_spec((1, NSPEC)),
        ],
        out_shape=[
            jax.ShapeDtypeStruct((B, S, HID), jnp.float32),
            jax.ShapeDtypeStruct((B, S, NSPEC + NSHARED), jnp.float32),
            jax.ShapeDtypeStruct((1, NSPEC), jnp.float32),
            jax.ShapeDtypeStruct((1, NSPEC), jnp.float32),
        ],
    )(h.reshape(B, S, HID), L['qkv_w'], L['qkv_b'].reshape(1, -1),
      L['out_w'], L['out_b'].reshape(1, -1),
      L['n1_g'].reshape(1, -1), L['n1_b'].reshape(1, -1),
      L['sp_router'], L['sh_router'])
    return y.reshape(T, HID), g.reshape(T, NSPEC + NSHARED), oh, rp


def _run_dense_experts(y, w1, b1, w2, b2, gcols):
    E = w1.shape[0]
    return pl.pallas_call(
        _moe_dense_kernel,
        grid=(E,),
        in_specs=[
            _full_spec((T, HID)),
            pl.BlockSpec((1, HID, FFN), lambda e: (e, 0, 0)),
            pl.BlockSpec((1, 1, FFN), lambda e: (e, 0, 0)),
            pl.BlockSpec((1, FFN, HID), lambda e: (e, 0, 0)),
            pl.BlockSpec((1, 1, HID), lambda e: (e, 0, 0)),
            pl.BlockSpec((1, T, 1), lambda e: (e, 0, 0)),
        ],
        out_specs=_full_spec((T, HID)),
        out_shape=jax.ShapeDtypeStruct((T, HID), jnp.float32),
    )(y, w1, b1.reshape(E, 1, -1), w2, b2.reshape(E, 1, -1), gcols)


def _run_combine(y, spec, shar, L):
    return pl.pallas_call(
        _combine_kernel,
        out_shape=jax.ShapeDtypeStruct((T, HID), jnp.float32),
    )(y, spec, shar, L['mn_g'].reshape(1, -1), L['mn_b'].reshape(1, -1),
      L['n2_g'].reshape(1, -1), L['n2_b'].reshape(1, -1))


def _run_head(h, p, oh, rp):
    return pl.pallas_call(
        _head_kernel,
        out_shape=[
            jax.ShapeDtypeStruct((T, EMB), jnp.float32),
            jax.ShapeDtypeStruct((B, NCLS), jnp.float32),
            jax.ShapeDtypeStruct((1, 1), jnp.float32),
        ],
    )(h, p['recon_w'], p['recon_b'].reshape(1, -1),
      p['cls_w'], p['cls_b'].reshape(1, -1), oh, rp)


def kernel(x, params):
    nH, nW = 128 // P, 128 // P
    t = x.reshape(B, IN_CH, nH, P, nW, P).transpose(0, 1, 2, 4, 3, 5)
    t = t.reshape(B, IN_CH, nH * nW, P, P).transpose(0, 2, 1, 3, 4)
    t = t.reshape(T, IN_CH * P * P)

    h = _run_embed(t, params)
    oh_list, rp_list = [], []
    for L in params['layers']:
        y, g, oh, rp = _run_attn(h, L)
        spec_cols = g[:, :NSPEC].T.reshape(NSPEC, T, 1)
        shar_cols = g[:, NSPEC:].T.reshape(NSHARED, T, 1)
        spec = _run_dense_experts(y, L['sp_fc1_w'], L['sp_fc1_b'],
                                  L['sp_fc2_w'], L['sp_fc2_b'], spec_cols)
        shar = _run_dense_experts(y, L['sh_fc1_w'], L['sh_fc1_b'],
                                  L['sh_fc2_w'], L['sh_fc2_b'], shar_cols)
        h = _run_combine(y, spec, shar, L)
        oh_list.append(oh)
        rp_list.append(rp)

    recon, logits, auxm = _run_head(
        h, params, jnp.concatenate(oh_list, 0), jnp.concatenate(rp_list, 0))
    aux = ALPHA * NSPEC * auxm.reshape(())
    return logits, recon.reshape(B, S, EMB), aux


# speedup vs baseline: 1.2155x; 1.2155x over previous
import math
import numpy as np
import jax
import jax.numpy as jnp
from jax.experimental import pallas as pl
from jax.experimental.pallas import tpu as pltpu

IN_CH = 5; P = 8; SFH = 64; EMB = 128; HID = 512; HEADS = 4; FFN = 2048
LAYERS = 2; NSPEC = 6; TOPK = 2; NSHARED = 2; NCLS = 2; ALPHA = 1e-4
B = 2; S = 256; T = B * S; DH = HID // HEADS
F2 = FFN // 2
NSTEPS = 2 + 18 * LAYERS


def _pos_encoding_np():
    pos = np.arange(S, dtype=np.float32)[:, None]
    div = np.exp(np.arange(0, EMB, 2, dtype=np.float32) * (-math.log(10000.0) / EMB))
    pe = np.zeros((S, EMB), np.float32)
    pe[:, 0::2] = np.sin(pos * div)
    pe[:, 1::2] = np.cos(pos * div)
    return np.tile(pe, (B, 1))


def _bdot(a, b):
    return jax.lax.dot_general(
        a.astype(jnp.bfloat16), b.astype(jnp.bfloat16),
        (((1,), (0,)), ((), ())), preferred_element_type=jnp.float32)


def _bdot_t(a, b):
    return jax.lax.dot_general(
        a.astype(jnp.bfloat16), b.astype(jnp.bfloat16),
        (((1,), (1,)), ((), ())), preferred_element_type=jnp.float32)


def _ln(x, g, b, eps=1e-5):
    m = x.mean(-1, keepdims=True)
    v = ((x - m) ** 2).mean(-1, keepdims=True)
    return (x - m) / jnp.sqrt(v + eps) * g + b


def _gelu(z):
    return 0.5 * z * (1.0 + jax.lax.erf(z * (1.0 / math.sqrt(2.0))))


N_GLOBAL = 12
N_PER_LAYER = 20
LKEYS = ['qkv_w', 'qkv_b', 'out_w', 'out_b', 'n1_g', 'n1_b', 'sp_router',
         'sh_router', 'mn_g', 'mn_b', 'n2_g', 'n2_b',
         'sp_fc1_w', 'sp_fc1_b', 'sp_fc2_w', 'sp_fc2_b',
         'sh_fc1_w', 'sh_fc1_b', 'sh_fc2_w', 'sh_fc2_b']


def _mega_kernel(*refs):
    nin = N_GLOBAL + N_PER_LAYER * LAYERS
    (t_r, sf1, sfb1, sf2, sfb2, pos, pw, pb, rw, rb, cw, cb) = refs[:N_GLOBAL]
    layer_refs = [refs[N_GLOBAL + l * N_PER_LAYER: N_GLOBAL + (l + 1) * N_PER_LAYER]
                  for l in range(LAYERS)]
    recon_o, logits_o, aux_o = refs[nin:nin + 3]
    h_s, y_s, acc_s, g_s, oh_s, rp_s = refs[nin + 3:]

    s_id = pl.program_id(0)

    @pl.when(s_id == 0)
    def _():
        t = jnp.maximum(t_r[...] @ sf1[...] + sfb1[...], 0.0)
        t = jnp.maximum(t @ sf2[...] + sfb2[...], 0.0)
        t = t + pos[...]
        h_s[...] = t @ pw[...] + pb[...]

    for l in range(LAYERS):
        base = 1 + 18 * l
        (qkvw, qkvb, outw, outb, n1g, n1b, spr, shr, mng, mnb, n2g, n2b,
         spw1, spb1, spw2, spb2, shw1, shb1, shw2, shb2) = layer_refs[l]

        @pl.when(s_id == base)
        def _(l=l, qkvw=qkvw, qkvb=qkvb, outw=outw, outb=outb, n1g=n1g,
              n1b=n1b, spr=spr, shr=shr):
            oh_acc = None
            rp_acc = None
            scale = 1.0 / math.sqrt(DH)
            for b in range(B):
                x = h_s[b * S:(b + 1) * S, :]
                qkv = _bdot(x, qkvw[...]) + qkvb[...]
                outs = []
                for hd in range(HEADS):
                    q = qkv[:, hd * DH:(hd + 1) * DH]
                    k = qkv[:, HID + hd * DH: HID + (hd + 1) * DH]
                    v = qkv[:, 2 * HID + hd * DH: 2 * HID + (hd + 1) * DH]
                    sc = _bdot_t(q, k) * scale
                    a = jax.nn.softmax(sc, axis=-1)
                    outs.append(_bdot(a, v))
                o = jnp.concatenate(outs, axis=1)
                att = _bdot(o, outw[...]) + outb[...]
                y = _ln(x + att, n1g[...], n1b[...])
                y_s[b * S:(b + 1) * S, :] = y

                pr = jax.nn.softmax(y @ spr[...], axis=-1)
                p1 = jnp.full((S, 1), -1.0, jnp.float32)
                i1 = jnp.zeros((S, 1), jnp.int32)
                for e in range(NSPEC):
                    pe = pr[:, e:e + 1]
                    upd = pe > p1
                    i1 = jnp.where(upd, e, i1)
                    p1 = jnp.where(upd, pe, p1)
                p2 = jnp.full((S, 1), -1.0, jnp.float32)
                i2 = jnp.zeros((S, 1), jnp.int32)
                for e in range(NSPEC):
                    pe = pr[:, e:e + 1]
                    upd = (pe > p2) & (i1 != e)
                    i2 = jnp.where(upd, e, i2)
                    p2 = jnp.where(upd, pe, p2)
                den = p1 + p2 + 1e-9
                w1n = p1 / den
                w2n = p2 / den
                for e in range(NSPEC):
                    ge = jnp.where(i1 == e, w1n, 0.0) + \
                        jnp.where(i2 == e, w2n, 0.0)
                    g_s[e, b * S:(b + 1) * S, :] = ge
                sh_p = jax.nn.softmax(y @ shr[...], axis=-1)
                for e in range(NSHARED):
                    g_s[NSPEC + e, b * S:(b + 1) * S, :] = sh_p[:, e:e + 1]

                ohb = jnp.concatenate(
                    [jnp.sum(((i1 == e) | (i2 == e)).astype(jnp.float32),
                             axis=0, keepdims=True) for e in range(NSPEC)]
                    + [jnp.zeros((1, 2), jnp.float32)], axis=1)
                rpb = jnp.concatenate(
                    [jnp.sum(pr, axis=0, keepdims=True),
                     jnp.zeros((1, 2), jnp.float32)], axis=1)
                oh_acc = ohb if oh_acc is None else oh_acc + ohb
                rp_acc = rpb if rp_acc is None else rp_acc + rpb
            oh_s[l:l + 1, :] = oh_acc
            rp_s[l:l + 1, :] = rp_acc
            acc_s[...] = jnp.zeros((T, HID), jnp.float32)

        def _expert_step(w1r, b1r, w2r, b2r, start, gate_base):
            rel = s_id - start
            f0 = (rel % 2) == 0
            e = rel // 2
            z = _bdot(y_s[...], w1r[0]) + b1r[0]
            h1 = _gelu(z)
            part = _bdot(h1, w2r[0])
            g = g_s[pl.ds(gate_base + e, 1), :, :][0]
            bias = jnp.where(f0, 1.0, 0.0) * b2r[0]
            acc_s[...] += (part + bias) * g

        @pl.when((s_id >= base + 1) & (s_id <= base + 12))
        def _(spw1=spw1, spb1=spb1, spw2=spw2, spb2=spb2, base=base):
            _expert_step(spw1, spb1, spw2, spb2, base + 1, 0)

        @pl.when((s_id >= base + 13) & (s_id <= base + 16))
        def _(shw1=shw1, shb1=shb1, shw2=shw2, shb2=shb2, base=base):
            _expert_step(shw1, shb1, shw2, shb2, base + 13, NSPEC)

        @pl.when(s_id == base + 17)
        def _(mng=mng, mnb=mnb, n2g=n2g, n2b=n2b):
            y = y_s[...]
            m = _ln(y + acc_s[...], mng[...], mnb[...])
            h_s[...] = _ln(y + m, n2g[...], n2b[...])

    @pl.when(s_id == NSTEPS - 1)
    def _():
        h = h_s[...]
        recon_o[...] = h @ rw[...] + rb[...]
        pooled = jnp.concatenate(
            [jnp.mean(h[b * S:(b + 1) * S, :], axis=0, keepdims=True)
             for b in range(B)], axis=0)
        logits_o[...] = pooled @ cw[...] + cb[...]
        ohm = oh_s[...] / float(T)
        rpm = rp_s[...] / float(T)
        aux_o[...] = jnp.sum(ohm * rpm).reshape(1, 1)


def _c2(s):
    return (0, 0)


def _c3(s):
    return (0, 0, 0)


def _sp_maps(l):
    st = 2 + 18 * l

    def w1(s):
        j = jnp.clip(s - st, 0, 11)
        return (j // 2, 0, j % 2)

    def b1(s):
        j = jnp.clip(s - st, 0, 11)
        return (j // 2, 0, j % 2)

    def w2(s):
        j = jnp.clip(s - st, 0, 11)
        return (j // 2, j % 2, 0)

    def b2(s):
        j = jnp.clip(s - st, 0, 11)
        return (j // 2, 0, 0)
    return w1, b1, w2, b2


def _sh_maps(l):
    st = 14 + 18 * l

    def w1(s):
        j = jnp.clip(s - st, 0, 3)
        return (j // 2, 0, j % 2)

    def b1(s):
        j = jnp.clip(s - st, 0, 3)
        return (j // 2, 0, j % 2)

    def w2(s):
        j = jnp.clip(s - st, 0, 3)
        return (j // 2, j % 2, 0)

    def b2(s):
        j = jnp.clip(s - st, 0, 3)
        return (j // 2, 0, 0)
    return w1, b1, w2, b2


def kernel(x, params):
    nH, nW = 128 // P, 128 // P
    t = x.reshape(B, IN_CH, nH, P, nW, P).transpose(0, 1, 2, 4, 3, 5)
    t = t.reshape(B, IN_CH, nH * nW, P, P).transpose(0, 2, 1, 3, 4)
    t = t.reshape(T, IN_CH * P * P)
    pos = jnp.asarray(_pos_encoding_np())
    p = params

    operands = [t, p['sf_w1'], p['sf_b1'].reshape(1, -1), p['sf_w2'],
                p['sf_b2'].reshape(1, -1), pos, p['proj_w'],
                p['proj_b'].reshape(1, -1), p['recon_w'],
                p['recon_b'].reshape(1, -1), p['cls_w'],
                p['cls_b'].reshape(1, -1)]
    in_specs = [
        pl.BlockSpec((T, IN_CH * P * P), _c2),
        pl.BlockSpec((IN_CH * P * P, SFH), _c2),
        pl.BlockSpec((1, SFH), _c2),
        pl.BlockSpec((SFH, EMB), _c2),
        pl.BlockSpec((1, EMB), _c2),
        pl.BlockSpec((T, EMB), _c2),
        pl.BlockSpec((EMB, HID), _c2),
        pl.BlockSpec((1, HID), _c2),
        pl.BlockSpec((HID, EMB), _c2),
        pl.BlockSpec((1, EMB), _c2),
        pl.BlockSpec((HID, NCLS), _c2),
        pl.BlockSpec((1, NCLS), _c2),
    ]
    for l, L in enumerate(p['layers']):
        spm = _sp_maps(l)
        shm = _sh_maps(l)
        operands += [
            L['qkv_w'], L['qkv_b'].reshape(1, -1), L['out_w'],
            L['out_b'].reshape(1, -1), L['n1_g'].reshape(1, -1),
            L['n1_b'].reshape(1, -1), L['sp_router'], L['sh_router'],
            L['mn_g'].reshape(1, -1), L['mn_b'].reshape(1, -1),
            L['n2_g'].reshape(1, -1), L['n2_b'].reshape(1, -1),
            L['sp_fc1_w'], L['sp_fc1_b'].reshape(NSPEC, 1, FFN),
            L['sp_fc2_w'], L['sp_fc2_b'].reshape(NSPEC, 1, HID),
            L['sh_fc1_w'], L['sh_fc1_b'].reshape(NSHARED, 1, FFN),
            L['sh_fc2_w'], L['sh_fc2_b'].reshape(NSHARED, 1, HID),
        ]
        in_specs += [
            pl.BlockSpec((HID, 3 * HID), _c2),
            pl.BlockSpec((1, 3 * HID), _c2),
            pl.BlockSpec((HID, HID), _c2),
            pl.BlockSpec((1, HID), _c2),
            pl.BlockSpec((1, HID), _c2),
            pl.BlockSpec((1, HID), _c2),
            pl.BlockSpec((HID, NSPEC), _c2),
            pl.BlockSpec((HID, NSHARED), _c2),
            pl.BlockSpec((1, HID), _c2),
            pl.BlockSpec((1, HID), _c2),
            pl.BlockSpec((1, HID), _c2),
            pl.BlockSpec((1, HID), _c2),
            pl.BlockSpec((1, HID, F2), spm[0]),
            pl.BlockSpec((1, 1, F2), spm[1]),
            pl.BlockSpec((1, F2, HID), spm[2]),
            pl.BlockSpec((1, 1, HID), spm[3]),
            pl.BlockSpec((1, HID, F2), shm[0]),
            pl.BlockSpec((1, 1, F2), shm[1]),
            pl.BlockSpec((1, F2, HID), shm[2]),
            pl.BlockSpec((1, 1, HID), shm[3]),
        ]

    recon, logits, auxm = pl.pallas_call(
        _mega_kernel,
        grid=(NSTEPS,),
        in_specs=in_specs,
        out_specs=[
            pl.BlockSpec((T, EMB), _c2),
            pl.BlockSpec((B, NCLS), _c2),
            pl.BlockSpec((1, 1), _c2),
        ],
        out_shape=[
            jax.ShapeDtypeStruct((T, EMB), jnp.float32),
            jax.ShapeDtypeStruct((B, NCLS), jnp.float32),
            jax.ShapeDtypeStruct((1, 1), jnp.float32),
        ],
        scratch_shapes=[
            pltpu.VMEM((T, HID), jnp.float32),
            pltpu.VMEM((T, HID), jnp.float32),
            pltpu.VMEM((T, HID), jnp.float32),
            pltpu.VMEM((NSPEC + NSHARED, T, 1), jnp.float32),
            pltpu.VMEM((LAYERS, 8), jnp.float32),
            pltpu.VMEM((LAYERS, 8), jnp.float32),
        ],
    )(*operands)

    aux = ALPHA * NSPEC * auxm.reshape(())
    return logits, recon.reshape(B, S, EMB), aux
